# Initial kernel scaffold; baseline (speedup 1.0000x reference)
import jax
import jax.numpy as jnp
from jax.experimental import pallas as pl


def kernel(x, edge_index, W1, b1, W2, b2):
    raise NotImplementedError("write your pallas kernel here")


# speedup vs baseline: 3.5546x; 3.5546x over previous
import functools

import jax
import jax.numpy as jnp
from jax import lax
from jax.experimental import pallas as pl
from jax.experimental.pallas import tpu as pltpu
from jax.experimental.pallas import tpu_sc as plsc

N_NODES = 10000
N_EDGES = 160000
F = 256
H = 128
NC = 2
NS = 16
CHUNK = 128
GROUPS = 79
E_PAD = NS * GROUPS * CHUNK
AGG_ROWS = 10240
ZROWS = AGG_ROWS // NS


def _mm1_body(x_ref, w_ref, b_ref, out_ref):
    acc = jnp.dot(x_ref[...], w_ref[...], preferred_element_type=jnp.float32)
    acc = jnp.maximum(acc + b_ref[...], 0.0)
    out_ref[0] = acc[:, :H]
    out_ref[1] = acc[:, H:]


def _mm1(x, w1, b1):
    bm = 2000
    grid = (N_NODES // bm,)
    return pl.pallas_call(
        _mm1_body,
        grid=grid,
        in_specs=[
            pl.BlockSpec((bm, F), lambda i: (i, 0)),
            pl.BlockSpec((F, F), lambda i: (0, 0)),
            pl.BlockSpec((1, F), lambda i: (0, 0)),
        ],
        out_specs=pl.BlockSpec((2, bm, H), lambda i: (0, i, 0)),
        out_shape=jax.ShapeDtypeStruct((2, N_NODES, H), jnp.float32),
    )(x, w1, b1)


def _mm2_body(a_ref, w_ref, b_ref, out_ref):
    acc = jnp.dot(a_ref[0], w_ref[:H, :], preferred_element_type=jnp.float32)
    acc += jnp.dot(a_ref[1], w_ref[H:, :], preferred_element_type=jnp.float32)
    out_ref[...] = jnp.maximum(acc + b_ref[...], 0.0)


def _mm2(agg2, w2, b2):
    bm = 2000
    grid = (N_NODES // bm,)
    return pl.pallas_call(
        _mm2_body,
        grid=grid,
        in_specs=[
            pl.BlockSpec((2, bm, H), lambda i: (0, i, 0)),
            pl.BlockSpec((F, F), lambda i: (0, 0)),
            pl.BlockSpec((1, F), lambda i: (0, 0)),
        ],
        out_specs=pl.BlockSpec((bm, F), lambda i: (i, 0)),
        out_shape=jax.ShapeDtypeStruct((N_NODES, F), jnp.float32),
    )(agg2, w2, b2)


def _sc_body(h_hbm, src_hbm, dst_hbm, z_hbm, agg_hbm,
             agg_sh, rows_v, sidx_v, didx_v, sem):
    c = lax.axis_index("c")
    s = lax.axis_index("s")

    pltpu.sync_copy(z_hbm, agg_sh.at[pl.ds(s * ZROWS, ZROWS)])
    plsc.subcore_barrier()

    base = s * (GROUPS * CHUNK)
    off = c * N_NODES

    def body(g, carry):
        ebase = base + g * CHUNK
        pltpu.sync_copy(src_hbm.at[pl.ds(ebase, CHUNK)], sidx_v)
        pltpu.sync_copy(dst_hbm.at[pl.ds(ebase, CHUNK)], didx_v)
        for k in range(CHUNK // 16):
            sl = pl.ds(k * 16, 16)
            sidx_v[sl] = sidx_v[sl] + off
        pltpu.async_copy(h_hbm.at[sidx_v], rows_v, sem).wait()
        pltpu.sync_copy(rows_v, agg_sh.at[didx_v], add=True)
        return carry

    lax.fori_loop(0, GROUPS, body, 0)
    plsc.subcore_barrier()

    pltpu.sync_copy(agg_sh.at[pl.ds(s * ZROWS, ZROWS)],
                    agg_hbm.at[pl.ds(c * AGG_ROWS + s * ZROWS, ZROWS)])


_sc_segsum = functools.partial(
    pl.kernel,
    out_type=jax.ShapeDtypeStruct((NC * AGG_ROWS, H), jnp.float32),
    mesh=plsc.VectorSubcoreMesh(core_axis_name="c", subcore_axis_name="s"),
    scratch_types=[
        pltpu.VMEM_SHARED((AGG_ROWS, H), jnp.float32),
        pltpu.VMEM((CHUNK, H), jnp.float32),
        pltpu.VMEM((CHUNK,), jnp.int32),
        pltpu.VMEM((CHUNK,), jnp.int32),
        pltpu.SemaphoreType.DMA,
    ],
)(_sc_body)


def kernel(x, edge_index, W1, b1, W2, b2):
    src = edge_index[0].astype(jnp.int32)
    dst = edge_index[1].astype(jnp.int32)
    pad = E_PAD - N_EDGES
    srcp = jnp.concatenate([src, jnp.zeros((pad,), jnp.int32)])
    dstp = jnp.concatenate([dst, jnp.full((pad,), N_NODES, jnp.int32)])
    zeros = jnp.zeros((ZROWS, H), jnp.float32)

    h2 = _mm1(x, W1, b1.reshape(1, F))
    agg_flat = _sc_segsum(h2.reshape(NC * N_NODES, H), srcp, dstp, zeros)
    return _mm2(agg_flat.reshape(NC, AGG_ROWS, H), W2, b2.reshape(1, F))
